# baseline (speedup 1.0000x reference)
import functools

import jax
import jax.numpy as jnp
from jax import lax
from jax.experimental import pallas as pl
from jax.experimental.pallas import tpu as pltpu
from jax.experimental.pallas import tpu_sc as plsc

N = 10000
E = 320000
D = 128
G = 64
NPAD = 10240
NC, NS = 2, 16
BLK = 1000
GRID = N // BLK

_f32 = jnp.float32


def _dot_t(a, b):
    return lax.dot_general(a, b, (((1,), (1,)), ((), ())),
                           preferred_element_type=_f32)


_MESH = plsc.VectorSubcoreMesh(core_axis_name="c", subcore_axis_name="s")
EPT = E // (NC * NS)
CH = 88
NFULL = 113
EMAIN = NFULL * CH
REM = EPT - EMAIN
CHD = 128
NFULLD = 78
EMAIND = NFULLD * CHD
REMD = EPT - EMAIND
RPT = NPAD // NS
NSLOT = 4
IR = 8
UNROLL = 8
NMAIN = 13
EPIL = 3 + NMAIN * UNROLL


_AGG_SCRATCH = (
    [pltpu.VMEM((REM,), jnp.int32),
     pltpu.VMEM((REM,), jnp.int32),
     pltpu.VMEM_SHARED((NPAD, D), _f32)]
    + [pltpu.VMEM((CH, D), _f32)] * NSLOT
    + [pltpu.VMEM((CH,), jnp.int32)] * IR
    + [pltpu.VMEM((CH,), jnp.int32)] * IR
    + [pltpu.SemaphoreType.DMA] * (2 * NSLOT + IR + 1)
)


@functools.partial(
    pl.kernel, mesh=_MESH,
    out_type=jax.ShapeDtypeStruct((NC, NPAD, D), _f32),
    scratch_types=_AGG_SCRATCH,
)
def _agg(z_hbm, ei_hbm, zer_hbm, out_hbm,
         sremv, dremv, acc,
         r0, r1, r2, r3,
         e0, e1, e2, e3, e4, e5, e6, e7,
         d0, d1, d2, d3, d4, d5, d6, d7,
         g0, g1, g2, g3, s0, s1, s2, s3,
         i0, i1, i2, i3, i4, i5, i6, i7, rsem):
    rows = (r0, r1, r2, r3)
    srci = (e0, e1, e2, e3, e4, e5, e6, e7)
    dsti = (d0, d1, d2, d3, d4, d5, d6, d7)
    gsem = (g0, g1, g2, g3)
    ssem = (s0, s1, s2, s3)
    isem = (i0, i1, i2, i3, i4, i5, i6, i7)
    c = lax.axis_index("c")
    s = lax.axis_index("s")
    wid = c * NS + s
    ebase = wid * EMAIN
    rbase = NC * NS * EMAIN + wid * REM

    def fire_idx(k, u):
        pltpu.async_copy(ei_hbm.at[pl.ds(ebase + k * CH, CH)],
                         srci[u], isem[u])
        pltpu.async_copy(ei_hbm.at[pl.ds(E + ebase + k * CH, CH)],
                         dsti[u], isem[u])

    def fire_gather(b, u):
        pltpu.make_async_copy(ei_hbm.at[pl.ds(0, CH)], srci[u],
                              isem[u]).wait()
        pltpu.make_async_copy(ei_hbm.at[pl.ds(0, CH)], dsti[u],
                              isem[u]).wait()
        pltpu.async_copy(z_hbm.at[srci[u]], rows[b], gsem[b])

    def wait_gather(b):
        pltpu.make_async_copy(z_hbm.at[srci[0]], rows[b], gsem[b]).wait()

    def fire_scatter(b, u):
        pltpu.async_copy(rows[b], acc.at[dsti[u]], ssem[b], add=True)

    def drain_scatter(b):
        pltpu.make_async_copy(rows[b], acc.at[dsti[0]], ssem[b]).wait()

    for k in range(5):
        fire_idx(k, k)
    for k in range(3):
        fire_gather(k, k)
    pltpu.sync_copy(ei_hbm.at[pl.ds(rbase, REM)], sremv)
    pltpu.sync_copy(ei_hbm.at[pl.ds(E + rbase, REM)], dremv)
    pltpu.sync_copy(zer_hbm, acc.at[pl.ds(s * RPT, RPT)])
    plsc.subcore_barrier()

    for k in range(3):
        wait_gather(k)
        fire_scatter(k, k)
        if k >= 1:
            drain_scatter(k - 1)
        fire_gather((k + 3) % NSLOT, k + 3)
        fire_idx(k + 5, k + 5 if k + 5 < IR else k + 5 - IR)

    def step(g, carry):
        for u in range(UNROLL):
            k = 3 + g * UNROLL + u
            b = (3 + u) % NSLOT
            bi = (3 + u) % IR
            wait_gather(b)
            fire_scatter(b, bi)
            drain_scatter((b + 3) % NSLOT)
            fire_gather((b + 3) % NSLOT, (bi + 3) % IR)
            fire_idx(k + 5, (bi + 5) % IR)
        return carry

    lax.fori_loop(0, NMAIN, step, 0)

    for k in range(EPIL, NFULL):
        b = k % NSLOT
        bi = k % IR
        wait_gather(b)
        fire_scatter(b, bi)
        drain_scatter((b + 3) % NSLOT)
        if k + 3 < NFULL:
            fire_gather((b + 3) % NSLOT, (bi + 3) % IR)
        if k + 5 < NFULL:
            fire_idx(k + 5, (bi + 5) % IR)
    drain_scatter((NFULL - 1) % NSLOT)
    rrem = r0.at[pl.ds(0, REM)]
    pltpu.async_copy(z_hbm.at[sremv], rrem, rsem)
    pltpu.make_async_copy(z_hbm.at[sremv], rrem, rsem).wait()
    pltpu.sync_copy(rrem, acc.at[dremv], add=True)

    plsc.subcore_barrier()
    pltpu.sync_copy(acc.at[pl.ds(s * RPT, RPT)],
                    out_hbm.at[c, pl.ds(s * RPT, RPT)])


@functools.partial(
    pl.kernel, mesh=_MESH,
    out_type=jax.ShapeDtypeStruct((NC, NPAD, D), _f32),
    scratch_types=(
        [pltpu.VMEM((REMD,), jnp.int32), pltpu.VMEM((CHD, D), _f32),
         pltpu.VMEM_SHARED((NPAD, D), _f32)]
        + [pltpu.VMEM((CHD,), jnp.int32)] * 2
        + [pltpu.SemaphoreType.DMA] * 4
    ),
)
def _deg(ei_hbm, one_hbm, zer_hbm, out_hbm,
         dremv, ones, acc, d0, d1, s0, s1, i0, i1):
    c = lax.axis_index("c")
    s = lax.axis_index("s")
    wid = c * NS + s
    ebase = wid * EMAIND
    rbase = NC * NS * EMAIND + wid * REMD
    dsti = (d0, d1)
    sems = (s0, s1)
    isem = (i0, i1)

    def fire_idx(k, u):
        pltpu.async_copy(ei_hbm.at[pl.ds(E + ebase + k * CHD, CHD)],
                         dsti[u], isem[u])

    def wait_idx(u):
        pltpu.make_async_copy(ei_hbm.at[pl.ds(0, CHD)], dsti[u],
                              isem[u]).wait()

    def drain_scatter(u):
        pltpu.make_async_copy(ones, acc.at[dsti[u]], sems[u]).wait()

    fire_idx(0, 0)
    fire_idx(1, 1)
    pltpu.sync_copy(ei_hbm.at[pl.ds(E + rbase, REMD)], dremv)
    pltpu.sync_copy(one_hbm, ones)
    pltpu.sync_copy(zer_hbm, acc.at[pl.ds(s * RPT, RPT)])
    plsc.subcore_barrier()

    for k in range(2):
        wait_idx(k)
        pltpu.async_copy(ones, acc.at[dsti[k]], sems[k], add=True)

    def step(g, carry):
        for u in range(2):
            k = 2 + g * 2 + u
            drain_scatter(u)
            fire_idx(k, u)
            wait_idx(u)
            pltpu.async_copy(ones, acc.at[dsti[u]], sems[u], add=True)
        return carry

    lax.fori_loop(0, (NFULLD - 2) // 2, step, 0)
    pltpu.sync_copy(ones.at[pl.ds(0, REMD)], acc.at[dremv], add=True)
    for u in range(2):
        drain_scatter(u)

    plsc.subcore_barrier()
    pltpu.sync_copy(acc.at[pl.ds(s * RPT, RPT)],
                    out_hbm.at[c, pl.ds(s * RPT, RPT)])


def _tc0_body(x_ref, w_ref, o_ref):
    o_ref[...] = _dot_t(x_ref[...], w_ref[...])


def _tc_r_body(hin_ref, wr_ref, b_ref, o_ref):
    o_ref[...] = _dot_t(hin_ref[...], wr_ref[...]) + b_ref[...]


def _tc_first_body(p_ref, pd_ref, r_ref, wl_ref, h_ref, z_ref, inv_ref):
    sm = p_ref[0] + p_ref[1]
    deg = pd_ref[0][:, 0:1] + pd_ref[1][:, 0:1]
    inv1 = 1.0 / jnp.maximum(deg, 1.0)
    h = jnp.maximum(sm * jnp.broadcast_to(inv1, (BLK, D)) + r_ref[...], 0.0)
    h_ref[...] = h
    z_ref[...] = _dot_t(h, wl_ref[...])
    inv_ref[...] = jnp.broadcast_to(inv1, (BLK, 8))


def _tc_mid_body(p_ref, inv_ref, r_ref, wl_ref, h_ref, z_ref):
    sm = p_ref[0] + p_ref[1]
    inv = jnp.broadcast_to(inv_ref[...][:, 0:1], (BLK, D))
    h = jnp.maximum(sm * inv + r_ref[...], 0.0)
    h_ref[...] = h
    z_ref[...] = _dot_t(h, wl_ref[...])


def _tc_last_body(p_ref, inv_ref, r_ref, bat_ref,
                  wf1_ref, bf1_ref, wf2_ref, bf2_ref, o_ref, acc_s, cnt_s):
    i = pl.program_id(0)
    sm = p_ref[0] + p_ref[1]
    inv = jnp.broadcast_to(inv_ref[...][:, 0:1], (BLK, D))
    h = jnp.maximum(sm * inv + r_ref[...], 0.0)
    bvec = bat_ref[0, 0, :]
    gid = lax.broadcasted_iota(jnp.int32, (G, BLK), 0)
    oneh = (bvec[None, :] == gid).astype(_f32)
    psum = lax.dot_general(oneh, h, (((1,), (0,)), ((), ())),
                           preferred_element_type=_f32)
    pcnt = jnp.broadcast_to(jnp.sum(oneh, axis=1, keepdims=True), (G, D))

    @pl.when(i == 0)
    def _():
        acc_s[...] = psum
        cnt_s[...] = pcnt

    @pl.when(i > 0)
    def _():
        acc_s[...] += psum
        cnt_s[...] += pcnt

    @pl.when(i == GRID - 1)
    def _():
        pooled = acc_s[...] / jnp.maximum(cnt_s[...], 1.0)
        e = _dot_t(pooled, wf1_ref[...]) + bf1_ref[...]
        o_ref[...] = _dot_t(e, wf2_ref[...]) + bf2_ref[...]


def _full(shape):
    return pl.BlockSpec(shape, lambda i: tuple(0 for _ in shape))


def _rows(w):
    return pl.BlockSpec((BLK, w), lambda i: (i, 0))


def _prt(w):
    return pl.BlockSpec((NC, BLK, w), lambda i: (0, i, 0))


def _tc0(x, w):
    return pl.pallas_call(
        _tc0_body, grid=(GRID,),
        in_specs=[_rows(D), _full((D, D))],
        out_specs=_rows(D),
        out_shape=jax.ShapeDtypeStruct((N, D), _f32),
    )(x, w)


def _tc_r(hin, wr, b):
    return pl.pallas_call(
        _tc_r_body, grid=(GRID,),
        in_specs=[_rows(D), _full((D, D)), _full((1, D))],
        out_specs=_rows(D),
        out_shape=jax.ShapeDtypeStruct((N, D), _f32),
    )(hin, wr, b)


def _tc_first(p, pd, r, wl):
    return pl.pallas_call(
        _tc_first_body, grid=(GRID,),
        in_specs=[_prt(D), _prt(D), _rows(D), _full((D, D))],
        out_specs=[_rows(D), _rows(D), _rows(8)],
        out_shape=[jax.ShapeDtypeStruct((N, D), _f32)] * 2
        + [jax.ShapeDtypeStruct((N, 8), _f32)],
    )(p, pd, r, wl)


def _tc_mid(p, inv, r, wl):
    return pl.pallas_call(
        _tc_mid_body, grid=(GRID,),
        in_specs=[_prt(D), _rows(8), _rows(D), _full((D, D))],
        out_specs=[_rows(D), _rows(D)],
        out_shape=[jax.ShapeDtypeStruct((N, D), _f32)] * 2,
    )(p, inv, r, wl)


def _tc_last(p, inv, r, bat3, wf1, bf1, wf2, bf2):
    return pl.pallas_call(
        _tc_last_body, grid=(GRID,),
        in_specs=[_prt(D), _rows(8), _rows(D),
                  pl.BlockSpec((1, 1, BLK), lambda i: (i, 0, 0)),
                  _full((D, D)), _full((1, D)), _full((10, D)),
                  _full((1, 10))],
        out_specs=_full((G, 10)),
        out_shape=jax.ShapeDtypeStruct((G, 10), _f32),
        scratch_shapes=[pltpu.VMEM((G, D), _f32), pltpu.VMEM((G, D), _f32)],
    )(p, inv, r, bat3, wf1, bf1, wf2, bf2)


def kernel(x, edge_index, batch, num_graphs,
           Wl1, bl1, Wr1, Wl2, bl2, Wr2, Wl3, bl3, Wr3, Wl4, bl4, Wr4,
           Wf1, bf1, Wf2, bf2):
    ei = edge_index.reshape(2 * E)
    zer = jnp.zeros((RPT, D), _f32)
    one = jnp.ones((CHD, D), _f32)
    bat3 = batch.reshape(GRID, 1, BLK)

    pdeg = _deg(ei, one, zer)
    z1 = _tc0(x, Wl1)
    p1 = _agg(z1, ei, zer)
    r1 = _tc_r(x, Wr1, bl1.reshape(1, D))
    h1, z2, inv = _tc_first(p1, pdeg, r1, Wl2)

    p2 = _agg(z2, ei, zer)
    r2 = _tc_r(h1, Wr2, bl2.reshape(1, D))
    h2, z3 = _tc_mid(p2, inv, r2, Wl3)

    p3 = _agg(z3, ei, zer)
    r3 = _tc_r(h2, Wr3, bl3.reshape(1, D))
    h3, z4 = _tc_mid(p3, inv, r3, Wl4)

    p4 = _agg(z4, ei, zer)
    r4 = _tc_r(h3, Wr4, bl4.reshape(1, D))
    out = _tc_last(p4, inv, r4, bat3,
                   Wf1, bf1.reshape(1, D), Wf2, bf2.reshape(1, 10))
    return out


# speedup vs baseline: 1.0122x; 1.0122x over previous
import functools

import jax
import jax.numpy as jnp
from jax import lax
from jax.experimental import pallas as pl
from jax.experimental.pallas import tpu as pltpu
from jax.experimental.pallas import tpu_sc as plsc

N = 10000
E = 320000
D = 128
G = 64
NPAD = 10240
NC, NS = 2, 16
BLK = 1000
GRID = N // BLK

_f32 = jnp.float32


def _dot_t(a, b):
    return lax.dot_general(a, b, (((1,), (1,)), ((), ())),
                           preferred_element_type=_f32)


_MESH = plsc.VectorSubcoreMesh(core_axis_name="c", subcore_axis_name="s")
EPT = E // (NC * NS)
CH = 88
NFULL = 113
EMAIN = NFULL * CH
REM = EPT - EMAIN
CHD = 128
NFULLD = 78
EMAIND = NFULLD * CHD
REMD = EPT - EMAIND
RPT = NPAD // NS
NSLOT = 4
IR = 8
UNROLL = 8
NMAIN = 13
EPIL = 3 + NMAIN * UNROLL


_AGG_SCRATCH = (
    [pltpu.VMEM((REM,), jnp.int32),
     pltpu.VMEM((REM,), jnp.int32),
     pltpu.VMEM_SHARED((NPAD, D), _f32)]
    + [pltpu.VMEM((CH, D), _f32)] * NSLOT
    + [pltpu.VMEM((CH,), jnp.int32)] * IR
    + [pltpu.VMEM((CH,), jnp.int32)] * IR
    + [pltpu.SemaphoreType.DMA] * (2 * NSLOT + IR + 1)
)


@functools.partial(
    pl.kernel, mesh=_MESH,
    out_type=jax.ShapeDtypeStruct((NC, NPAD, D), _f32),
    scratch_types=_AGG_SCRATCH,
)
def _agg(z_hbm, ei_hbm, zer_hbm, out_hbm,
         sremv, dremv, acc,
         r0, r1, r2, r3,
         e0, e1, e2, e3, e4, e5, e6, e7,
         d0, d1, d2, d3, d4, d5, d6, d7,
         g0, g1, g2, g3, s0, s1, s2, s3,
         i0, i1, i2, i3, i4, i5, i6, i7, rsem):
    rows = (r0, r1, r2, r3)
    srci = (e0, e1, e2, e3, e4, e5, e6, e7)
    dsti = (d0, d1, d2, d3, d4, d5, d6, d7)
    gsem = (g0, g1, g2, g3)
    ssem = (s0, s1, s2, s3)
    isem = (i0, i1, i2, i3, i4, i5, i6, i7)
    c = lax.axis_index("c")
    s = lax.axis_index("s")
    wid = c * NS + s
    ebase = wid * EMAIN
    rbase = NC * NS * EMAIN + wid * REM

    def fire_idx(k, u):
        pltpu.async_copy(ei_hbm.at[pl.ds(ebase + k * CH, CH)],
                         srci[u], isem[u])
        pltpu.async_copy(ei_hbm.at[pl.ds(E + ebase + k * CH, CH)],
                         dsti[u], isem[u])

    def fire_gather(b, u):
        pltpu.make_async_copy(ei_hbm.at[pl.ds(0, CH)], srci[u],
                              isem[u]).wait()
        pltpu.make_async_copy(ei_hbm.at[pl.ds(0, CH)], dsti[u],
                              isem[u]).wait()
        pltpu.async_copy(z_hbm.at[srci[u]], rows[b], gsem[b])

    def wait_gather(b):
        pltpu.make_async_copy(z_hbm.at[srci[0]], rows[b], gsem[b]).wait()

    def fire_scatter(b, u):
        pltpu.async_copy(rows[b], acc.at[dsti[u]], ssem[b], add=True)

    def drain_scatter(b):
        pltpu.make_async_copy(rows[b], acc.at[dsti[0]], ssem[b]).wait()

    for k in range(5):
        fire_idx(k, k)
    for k in range(3):
        fire_gather(k, k)
    pltpu.sync_copy(ei_hbm.at[pl.ds(rbase, REM)], sremv)
    pltpu.sync_copy(ei_hbm.at[pl.ds(E + rbase, REM)], dremv)
    pltpu.sync_copy(zer_hbm, acc.at[pl.ds(s * RPT, RPT)])
    plsc.subcore_barrier()

    for k in range(3):
        wait_gather(k)
        fire_scatter(k, k)
        if k >= 1:
            drain_scatter(k - 1)
        fire_gather((k + 3) % NSLOT, k + 3)
        fire_idx(k + 5, k + 5 if k + 5 < IR else k + 5 - IR)

    def step(g, carry):
        for u in range(UNROLL):
            k = 3 + g * UNROLL + u
            b = (3 + u) % NSLOT
            bi = (3 + u) % IR
            wait_gather(b)
            fire_scatter(b, bi)
            drain_scatter((b + 3) % NSLOT)
            fire_gather((b + 3) % NSLOT, (bi + 3) % IR)
            fire_idx(k + 5, (bi + 5) % IR)
        return carry

    lax.fori_loop(0, NMAIN, step, 0)

    for k in range(EPIL, NFULL):
        b = k % NSLOT
        bi = k % IR
        wait_gather(b)
        fire_scatter(b, bi)
        drain_scatter((b + 3) % NSLOT)
        if k + 3 < NFULL:
            fire_gather((b + 3) % NSLOT, (bi + 3) % IR)
        if k + 5 < NFULL:
            fire_idx(k + 5, (bi + 5) % IR)
    drain_scatter((NFULL - 1) % NSLOT)
    rrem = r0.at[pl.ds(0, REM)]
    pltpu.async_copy(z_hbm.at[sremv], rrem, rsem)
    pltpu.make_async_copy(z_hbm.at[sremv], rrem, rsem).wait()
    pltpu.sync_copy(rrem, acc.at[dremv], add=True)

    plsc.subcore_barrier()
    pltpu.sync_copy(acc.at[pl.ds(s * RPT, RPT)],
                    out_hbm.at[c, pl.ds(s * RPT, RPT)])


@functools.partial(
    pl.kernel, mesh=_MESH,
    out_type=jax.ShapeDtypeStruct((NC, NPAD, D), _f32),
    scratch_types=(
        [pltpu.VMEM((REMD,), jnp.int32), pltpu.VMEM((CHD, D), _f32),
         pltpu.VMEM_SHARED((NPAD, D), _f32)]
        + [pltpu.VMEM((CHD,), jnp.int32)] * 2
        + [pltpu.SemaphoreType.DMA] * 4
    ),
)
def _deg(ei_hbm, one_hbm, zer_hbm, out_hbm,
         dremv, ones, acc, d0, d1, s0, s1, i0, i1):
    c = lax.axis_index("c")
    s = lax.axis_index("s")
    wid = c * NS + s
    ebase = wid * EMAIND
    rbase = NC * NS * EMAIND + wid * REMD
    dsti = (d0, d1)
    sems = (s0, s1)
    isem = (i0, i1)

    def fire_idx(k, u):
        pltpu.async_copy(ei_hbm.at[pl.ds(E + ebase + k * CHD, CHD)],
                         dsti[u], isem[u])

    def wait_idx(u):
        pltpu.make_async_copy(ei_hbm.at[pl.ds(0, CHD)], dsti[u],
                              isem[u]).wait()

    def drain_scatter(u):
        pltpu.make_async_copy(ones, acc.at[dsti[u]], sems[u]).wait()

    fire_idx(0, 0)
    fire_idx(1, 1)
    pltpu.sync_copy(ei_hbm.at[pl.ds(E + rbase, REMD)], dremv)
    pltpu.sync_copy(one_hbm, ones)
    pltpu.sync_copy(zer_hbm, acc.at[pl.ds(s * RPT, RPT)])
    plsc.subcore_barrier()

    for k in range(2):
        wait_idx(k)
        pltpu.async_copy(ones, acc.at[dsti[k]], sems[k], add=True)

    def step(g, carry):
        for u in range(2):
            k = 2 + g * 2 + u
            drain_scatter(u)
            fire_idx(k, u)
            wait_idx(u)
            pltpu.async_copy(ones, acc.at[dsti[u]], sems[u], add=True)
        return carry

    lax.fori_loop(0, (NFULLD - 2) // 2, step, 0)
    pltpu.sync_copy(ones.at[pl.ds(0, REMD)], acc.at[dremv], add=True)
    for u in range(2):
        drain_scatter(u)

    plsc.subcore_barrier()
    pltpu.sync_copy(acc.at[pl.ds(s * RPT, RPT)],
                    out_hbm.at[c, pl.ds(s * RPT, RPT)])


def _tc0_body(x_ref, w_ref, o_ref):
    o_ref[...] = _dot_t(x_ref[...], w_ref[...])


def _tc_first_body(p_ref, pd_ref, x_ref, wr_ref, b_ref, wl_ref,
                   h_ref, z_ref, inv_ref):
    sm = p_ref[0] + p_ref[1]
    deg = pd_ref[0][:, 0:1] + pd_ref[1][:, 0:1]
    inv1 = 1.0 / jnp.maximum(deg, 1.0)
    h = jnp.maximum(sm * jnp.broadcast_to(inv1, (BLK, D))
                    + _dot_t(x_ref[...], wr_ref[...]) + b_ref[...], 0.0)
    h_ref[...] = h
    z_ref[...] = _dot_t(h, wl_ref[...])
    inv_ref[...] = jnp.broadcast_to(inv1, (BLK, 8))


def _tc_mid_body(p_ref, inv_ref, hin_ref, wr_ref, b_ref, wl_ref, h_ref, z_ref):
    sm = p_ref[0] + p_ref[1]
    inv = jnp.broadcast_to(inv_ref[...][:, 0:1], (BLK, D))
    h = jnp.maximum(sm * inv
                    + _dot_t(hin_ref[...], wr_ref[...]) + b_ref[...], 0.0)
    h_ref[...] = h
    z_ref[...] = _dot_t(h, wl_ref[...])


def _tc_last_body(p_ref, inv_ref, hin_ref, wr_ref, b_ref, bat_ref,
                  wf1_ref, bf1_ref, wf2_ref, bf2_ref, o_ref, acc_s, cnt_s):
    i = pl.program_id(0)
    sm = p_ref[0] + p_ref[1]
    inv = jnp.broadcast_to(inv_ref[...][:, 0:1], (BLK, D))
    h = jnp.maximum(sm * inv
                    + _dot_t(hin_ref[...], wr_ref[...]) + b_ref[...], 0.0)
    bvec = bat_ref[0, 0, :]
    gid = lax.broadcasted_iota(jnp.int32, (G, BLK), 0)
    oneh = (bvec[None, :] == gid).astype(_f32)
    psum = lax.dot_general(oneh, h, (((1,), (0,)), ((), ())),
                           preferred_element_type=_f32)
    pcnt = jnp.broadcast_to(jnp.sum(oneh, axis=1, keepdims=True), (G, D))

    @pl.when(i == 0)
    def _():
        acc_s[...] = psum
        cnt_s[...] = pcnt

    @pl.when(i > 0)
    def _():
        acc_s[...] += psum
        cnt_s[...] += pcnt

    @pl.when(i == GRID - 1)
    def _():
        pooled = acc_s[...] / jnp.maximum(cnt_s[...], 1.0)
        e = _dot_t(pooled, wf1_ref[...]) + bf1_ref[...]
        o_ref[...] = _dot_t(e, wf2_ref[...]) + bf2_ref[...]


def _full(shape):
    return pl.BlockSpec(shape, lambda i: tuple(0 for _ in shape))


def _rows(w):
    return pl.BlockSpec((BLK, w), lambda i: (i, 0))


def _prt(w):
    return pl.BlockSpec((NC, BLK, w), lambda i: (0, i, 0))


def _tc0(x, w):
    return pl.pallas_call(
        _tc0_body, grid=(GRID,),
        in_specs=[_rows(D), _full((D, D))],
        out_specs=_rows(D),
        out_shape=jax.ShapeDtypeStruct((N, D), _f32),
    )(x, w)


def _tc_first(p, pd, x, wr, b, wl):
    return pl.pallas_call(
        _tc_first_body, grid=(GRID,),
        in_specs=[_prt(D), _prt(D), _rows(D), _full((D, D)), _full((1, D)),
                  _full((D, D))],
        out_specs=[_rows(D), _rows(D), _rows(8)],
        out_shape=[jax.ShapeDtypeStruct((N, D), _f32)] * 2
        + [jax.ShapeDtypeStruct((N, 8), _f32)],
    )(p, pd, x, wr, b, wl)


def _tc_mid(p, inv, hin, wr, b, wl):
    return pl.pallas_call(
        _tc_mid_body, grid=(GRID,),
        in_specs=[_prt(D), _rows(8), _rows(D), _full((D, D)), _full((1, D)),
                  _full((D, D))],
        out_specs=[_rows(D), _rows(D)],
        out_shape=[jax.ShapeDtypeStruct((N, D), _f32)] * 2,
    )(p, inv, hin, wr, b, wl)


def _tc_last(p, inv, hin, wr, b, bat3, wf1, bf1, wf2, bf2):
    return pl.pallas_call(
        _tc_last_body, grid=(GRID,),
        in_specs=[_prt(D), _rows(8), _rows(D), _full((D, D)), _full((1, D)),
                  pl.BlockSpec((1, 1, BLK), lambda i: (i, 0, 0)),
                  _full((D, D)), _full((1, D)), _full((10, D)),
                  _full((1, 10))],
        out_specs=_full((G, 10)),
        out_shape=jax.ShapeDtypeStruct((G, 10), _f32),
        scratch_shapes=[pltpu.VMEM((G, D), _f32), pltpu.VMEM((G, D), _f32)],
    )(p, inv, hin, wr, b, bat3, wf1, bf1, wf2, bf2)


def kernel(x, edge_index, batch, num_graphs,
           Wl1, bl1, Wr1, Wl2, bl2, Wr2, Wl3, bl3, Wr3, Wl4, bl4, Wr4,
           Wf1, bf1, Wf2, bf2):
    ei = edge_index.reshape(2 * E)
    zer = jnp.zeros((RPT, D), _f32)
    one = jnp.ones((CHD, D), _f32)
    bat3 = batch.reshape(GRID, 1, BLK)

    pdeg = _deg(ei, one, zer)
    z1 = _tc0(x, Wl1)
    p1 = _agg(z1, ei, zer)
    h1, z2, inv = _tc_first(p1, pdeg, x, Wr1, bl1.reshape(1, D), Wl2)

    p2 = _agg(z2, ei, zer)
    h2, z3 = _tc_mid(p2, inv, h1, Wr2, bl2.reshape(1, D), Wl3)

    p3 = _agg(z3, ei, zer)
    h3, z4 = _tc_mid(p3, inv, h2, Wr3, bl3.reshape(1, D), Wl4)

    p4 = _agg(z4, ei, zer)
    out = _tc_last(p4, inv, h3, Wr4, bl4.reshape(1, D), bat3,
                   Wf1, bf1.reshape(1, D), Wf2, bf2.reshape(1, 10))
    return out
